# Initial kernel scaffold; baseline (speedup 1.0000x reference)
import jax
import jax.numpy as jnp
from jax.experimental import pallas as pl


def kernel(x, edge_index, W_enc, b_enc, W_h0, b_h0, W_h1, b_h1, W_h2, b_h2, W_dec, b_dec):
    raise NotImplementedError("write your pallas kernel here")


# speedup vs baseline: 8.2852x; 8.2852x over previous
import functools

import jax
import jax.numpy as jnp
from jax import lax
from jax.experimental import pallas as pl
from jax.experimental.pallas import tpu as pltpu
from jax.experimental.pallas import tpu_sc as plsc

N = 10000
NP = 10240
E = 160000
NS = 16
NC = 2
EPT = E // NS
KB = 80
NBLK = EPT // KB
ROWS_PT = NP // NS
BM = 1000


def _sc_agg(C):
    P = C // NC
    mesh = plsc.VectorSubcoreMesh(core_axis_name="c", subcore_axis_name="s")

    @functools.partial(
        pl.kernel,
        out_type=jax.ShapeDtypeStruct((C * NP, 128), jnp.float32),
        mesh=mesh,
        scratch_types=[
            pltpu.VMEM((NBLK, KB), jnp.int32),
            pltpu.VMEM((NBLK, KB), jnp.int32),
            pltpu.VMEM((KB, 128), jnp.float32),
            pltpu.VMEM((KB, 128), jnp.float32),
            pltpu.VMEM_SHARED((NP, 128), jnp.float32),
            pltpu.SemaphoreType.DMA,
            pltpu.SemaphoreType.DMA,
        ],
        compiler_params=pltpu.CompilerParams(use_tc_tiling_on_sc=False),
    )
    def k(y, src, dst, out, src_v, dst_v, buf_a, buf_b, acc, sem_a, sem_b):
        c = lax.axis_index("c")
        s = lax.axis_index("s")
        pltpu.sync_copy(src.at[s], src_v)
        pltpu.sync_copy(dst.at[s], dst_v)
        for p in range(P):
            base = (c * P + p) * NP
            pltpu.sync_copy(y.at[pl.ds(base + s * ROWS_PT, ROWS_PT)],
                            acc.at[pl.ds(s * ROWS_PT, ROWS_PT)])
            plsc.subcore_barrier()

            ytab = y.at[pl.ds(base, NP)]
            pltpu.async_copy(ytab.at[src_v.at[0]], buf_a, sem_a)

            def body(i, carry):
                j0 = 2 * i
                pltpu.make_async_copy(
                    ytab.at[src_v.at[0]], buf_a, sem_a).wait()
                pltpu.async_copy(ytab.at[src_v.at[j0 + 1]], buf_b, sem_b)
                pltpu.sync_copy(buf_a, acc.at[dst_v.at[j0]], add=True)
                pltpu.make_async_copy(
                    ytab.at[src_v.at[0]], buf_b, sem_b).wait()
                pltpu.async_copy(ytab.at[src_v.at[j0 + 2]], buf_a, sem_a)
                pltpu.sync_copy(buf_b, acc.at[dst_v.at[j0 + 1]], add=True)
                return carry

            lax.fori_loop(0, (NBLK - 1) // 2, body, 0)
            pltpu.make_async_copy(
                ytab.at[src_v.at[0]], buf_a, sem_a).wait()
            pltpu.sync_copy(buf_a, acc.at[dst_v.at[NBLK - 1]], add=True)
            plsc.subcore_barrier()

            pltpu.sync_copy(acc.at[pl.ds(s * ROWS_PT, ROWS_PT)],
                            out.at[pl.ds(base + s * ROWS_PT, ROWS_PT)])
            plsc.subcore_barrier()

    return k


def _sc_deg():
    mesh = plsc.VectorSubcoreMesh(core_axis_name="c", subcore_axis_name="s")

    @functools.partial(
        pl.kernel,
        out_type=jax.ShapeDtypeStruct((NP, 16), jnp.float32),
        mesh=mesh,
        scratch_types=[
            pltpu.VMEM((NBLK, KB), jnp.int32),
            pltpu.VMEM((KB, 16), jnp.float32),
            pltpu.VMEM((ROWS_PT, 16), jnp.float32),
            pltpu.VMEM_SHARED((NP, 16), jnp.float32),
        ],
        compiler_params=pltpu.CompilerParams(use_tc_tiling_on_sc=False),
    )
    def k(dst, out, dst_v, ones_v, rows_v, acc):
        c = lax.axis_index("c")
        s = lax.axis_index("s")

        @pl.when(c == 0)
        def _():
            pltpu.sync_copy(dst.at[s], dst_v)

            def fill_ones(i, carry):
                ones_v[i, :] = jnp.full((16,), 1.0, jnp.float32)
                return carry

            lax.fori_loop(0, KB, fill_ones, 0)

            def fill_rows(i, carry):
                rows_v[i, :] = jnp.full((16,), 1.0, jnp.float32)
                return carry

            lax.fori_loop(0, ROWS_PT, fill_rows, 0)
            pltpu.sync_copy(rows_v, acc.at[pl.ds(s * ROWS_PT, ROWS_PT)])
            plsc.subcore_barrier()

            def body(j, carry):
                pltpu.sync_copy(ones_v, acc.at[dst_v.at[j]], add=True)
                return carry

            lax.fori_loop(0, NBLK, body, 0)
            plsc.subcore_barrier()
            pltpu.sync_copy(acc.at[pl.ds(s * ROWS_PT, ROWS_PT)], rows_v)
            pltpu.sync_copy(rows_v, out.at[pl.ds(s * ROWS_PT, ROWS_PT)])

    return k


def _leaky(x):
    return jnp.where(x > 0, x, 0.01 * x)


def _tc_matmul_first(x, w, deg):
    k_in, d_out = w.shape
    ck, cn = k_in // 128, d_out // 128
    grid = (N // BM, cn, ck)

    def body(x_ref, w_ref, deg_ref, out_ref):
        kk = pl.program_id(2)
        contrib = jnp.dot(x_ref[...], w_ref[...],
                          preferred_element_type=jnp.float32)

        @pl.when(kk == 0)
        def _():
            out_ref[0] = jnp.zeros_like(out_ref[0])

        out_ref[0] += contrib

        @pl.when(kk == ck - 1)
        def _():
            out_ref[0] = out_ref[0] * lax.rsqrt(deg_ref[:, 0:1])

    return pl.pallas_call(
        body,
        grid=grid,
        in_specs=[
            pl.BlockSpec((BM, 128), lambda m, n, k: (m, k)),
            pl.BlockSpec((128, 128), lambda m, n, k: (k, n)),
            pl.BlockSpec((BM, 16), lambda m, n, k: (m, 0)),
        ],
        out_specs=pl.BlockSpec((1, BM, 128), lambda m, n, k: (n, m, 0)),
        out_shape=jax.ShapeDtypeStruct((cn, NP, 128), jnp.float32),
    )(x, w, deg)


def _tc_matmul(agg, w, b_prev, deg):
    k_in, d_out = w.shape
    ck, cn = k_in // 128, d_out // 128
    grid = (N // BM, cn, ck)

    def body(agg_ref, w_ref, b_ref, deg_ref, out_ref):
        kk = pl.program_id(2)
        dinv = lax.rsqrt(deg_ref[:, 0:1])
        h = _leaky(dinv * agg_ref[0] + b_ref[0, 0])
        contrib = jnp.dot(h, w_ref[...], preferred_element_type=jnp.float32)

        @pl.when(kk == 0)
        def _():
            out_ref[0] = jnp.zeros_like(out_ref[0])

        out_ref[0] += contrib

        @pl.when(kk == ck - 1)
        def _():
            out_ref[0] = out_ref[0] * dinv

    return pl.pallas_call(
        body,
        grid=grid,
        in_specs=[
            pl.BlockSpec((1, BM, 128), lambda m, n, k: (k, m, 0)),
            pl.BlockSpec((128, 128), lambda m, n, k: (k, n)),
            pl.BlockSpec((1, 1, 128), lambda m, n, k: (k, 0, 0)),
            pl.BlockSpec((BM, 16), lambda m, n, k: (m, 0)),
        ],
        out_specs=pl.BlockSpec((1, BM, 128), lambda m, n, k: (n, m, 0)),
        out_shape=jax.ShapeDtypeStruct((cn, NP, 128), jnp.float32),
    )(agg, w, b_prev.reshape(ck, 1, 128), deg)


def _tc_epilogue(agg, b, deg, d_out):
    cn = d_out // 128
    grid = (N // BM, cn)

    def body(agg_ref, b_ref, deg_ref, out_ref):
        out_ref[...] = lax.rsqrt(deg_ref[:, 0:1]) * agg_ref[0] + b_ref[0, 0]

    return pl.pallas_call(
        body,
        grid=grid,
        in_specs=[
            pl.BlockSpec((1, BM, 128), lambda m, n: (n, m, 0)),
            pl.BlockSpec((1, 1, 128), lambda m, n: (n, 0, 0)),
            pl.BlockSpec((BM, 16), lambda m, n: (m, 0)),
        ],
        out_specs=pl.BlockSpec((BM, 128), lambda m, n: (m, n)),
        out_shape=jax.ShapeDtypeStruct((N, d_out), jnp.float32),
    )(agg, b.reshape(cn, 1, 128), deg)


_deg_kernel = _sc_deg()
_agg4 = _sc_agg(4)
_agg2 = _sc_agg(2)


def kernel(x, edge_index, W_enc, b_enc, W_h0, b_h0, W_h1, b_h1, W_h2, b_h2,
           W_dec, b_dec):
    src = edge_index[0].reshape(NS, NBLK, KB)
    dst = edge_index[1].reshape(NS, NBLK, KB)

    deg = _deg_kernel(dst)

    y = _tc_matmul_first(x, W_enc, deg)
    agg = _agg4(y.reshape(4 * NP, 128), src, dst)
    y = _tc_matmul(agg.reshape(4, NP, 128), W_h0, b_enc, deg)
    agg = _agg4(y.reshape(4 * NP, 128), src, dst)
    y = _tc_matmul(agg.reshape(4, NP, 128), W_h1, b_h0, deg)
    agg = _agg4(y.reshape(4 * NP, 128), src, dst)
    y = _tc_matmul(agg.reshape(4, NP, 128), W_h2, b_h1, deg)
    agg = _agg4(y.reshape(4 * NP, 128), src, dst)
    y = _tc_matmul(agg.reshape(4, NP, 128), W_dec, b_h2, deg)
    agg = _agg2(y.reshape(2 * NP, 128), src, dst)
    return _tc_epilogue(agg.reshape(2, NP, 128), b_dec, deg, 256)


# speedup vs baseline: 9.1409x; 1.1033x over previous
import functools

import jax
import jax.numpy as jnp
from jax import lax
from jax.experimental import pallas as pl
from jax.experimental.pallas import tpu as pltpu
from jax.experimental.pallas import tpu_sc as plsc

N = 10000
NP = 10240
E = 160000
NS = 16
NC = 2
EPT = E // NS
KB = 112
NBLK = 90
EPAD = NBLK * KB - EPT
ROWS_PT = NP // NS
BM = 1000


def _sc_agg(C):
    P = C // NC
    mesh = plsc.VectorSubcoreMesh(core_axis_name="c", subcore_axis_name="s")

    @functools.partial(
        pl.kernel,
        out_type=jax.ShapeDtypeStruct((C * NP, 128), jnp.float32),
        mesh=mesh,
        scratch_types=[
            pltpu.VMEM((NBLK, KB), jnp.int32),
            pltpu.VMEM((NBLK, KB), jnp.int32),
            pltpu.VMEM((KB, 128), jnp.float32),
            pltpu.VMEM((KB, 128), jnp.float32),
            pltpu.VMEM_SHARED((NP, 128), jnp.float32),
            pltpu.SemaphoreType.DMA,
            pltpu.SemaphoreType.DMA,
        ],
        compiler_params=pltpu.CompilerParams(use_tc_tiling_on_sc=False),
    )
    def k(y, src, dst, out, src_v, dst_v, buf_a, buf_b, acc, sem_a, sem_b):
        c = lax.axis_index("c")
        s = lax.axis_index("s")
        pltpu.sync_copy(src.at[s], src_v)
        pltpu.sync_copy(dst.at[s], dst_v)
        for p in range(P):
            base = (c * P + p) * NP
            pltpu.sync_copy(y.at[pl.ds(base + s * ROWS_PT, ROWS_PT)],
                            acc.at[pl.ds(s * ROWS_PT, ROWS_PT)])
            plsc.subcore_barrier()

            ytab = y.at[pl.ds(base, NP)]
            pltpu.async_copy(ytab.at[src_v.at[0]], buf_a, sem_a)

            def body(i, carry):
                j0 = 2 * i
                pltpu.make_async_copy(
                    ytab.at[src_v.at[0]], buf_a, sem_a).wait()
                pltpu.async_copy(ytab.at[src_v.at[j0 + 1]], buf_b, sem_b)
                pltpu.sync_copy(buf_a, acc.at[dst_v.at[j0]], add=True)
                pltpu.make_async_copy(
                    ytab.at[src_v.at[0]], buf_b, sem_b).wait()
                pltpu.async_copy(ytab.at[src_v.at[j0 + 2]], buf_a, sem_a)
                pltpu.sync_copy(buf_b, acc.at[dst_v.at[j0 + 1]], add=True)
                return carry

            lax.fori_loop(0, (NBLK - 1) // 2, body, 0)
            pltpu.make_async_copy(
                ytab.at[src_v.at[0]], buf_a, sem_a).wait()
            pltpu.sync_copy(buf_a, acc.at[dst_v.at[NBLK - 1]], add=True)
            plsc.subcore_barrier()

            pltpu.sync_copy(acc.at[pl.ds(s * ROWS_PT, ROWS_PT)],
                            out.at[pl.ds(base + s * ROWS_PT, ROWS_PT)])
            plsc.subcore_barrier()

    return k


def _sc_deg():
    mesh = plsc.VectorSubcoreMesh(core_axis_name="c", subcore_axis_name="s")

    @functools.partial(
        pl.kernel,
        out_type=jax.ShapeDtypeStruct((NP, 16), jnp.float32),
        mesh=mesh,
        scratch_types=[
            pltpu.VMEM((NBLK, KB), jnp.int32),
            pltpu.VMEM((KB, 16), jnp.float32),
            pltpu.VMEM((ROWS_PT, 16), jnp.float32),
            pltpu.VMEM_SHARED((NP, 16), jnp.float32),
        ],
        compiler_params=pltpu.CompilerParams(use_tc_tiling_on_sc=False),
    )
    def k(dst, out, dst_v, ones_v, rows_v, acc):
        c = lax.axis_index("c")
        s = lax.axis_index("s")

        @pl.when(c == 0)
        def _():
            pltpu.sync_copy(dst.at[s], dst_v)

            def fill_ones(i, carry):
                ones_v[i, :] = jnp.full((16,), 1.0, jnp.float32)
                return carry

            lax.fori_loop(0, KB, fill_ones, 0)

            def fill_rows(i, carry):
                rows_v[i, :] = jnp.full((16,), 1.0, jnp.float32)
                return carry

            lax.fori_loop(0, ROWS_PT, fill_rows, 0)
            pltpu.sync_copy(rows_v, acc.at[pl.ds(s * ROWS_PT, ROWS_PT)])
            plsc.subcore_barrier()

            def body(j, carry):
                pltpu.sync_copy(ones_v, acc.at[dst_v.at[j]], add=True)
                return carry

            lax.fori_loop(0, NBLK, body, 0)
            plsc.subcore_barrier()
            pltpu.sync_copy(acc.at[pl.ds(s * ROWS_PT, ROWS_PT)], rows_v)
            pltpu.sync_copy(rows_v, out.at[pl.ds(s * ROWS_PT, ROWS_PT)])

    return k


def _leaky(x):
    return jnp.where(x > 0, x, 0.01 * x)


def _tc_matmul_first(x, w, deg):
    k_in, d_out = w.shape
    ck, cn = k_in // 128, d_out // 128
    grid = (N // BM, cn, ck)

    def body(x_ref, w_ref, deg_ref, out_ref):
        kk = pl.program_id(2)
        contrib = jnp.dot(x_ref[...], w_ref[...],
                          preferred_element_type=jnp.float32)

        @pl.when(kk == 0)
        def _():
            out_ref[0] = jnp.zeros_like(out_ref[0])

        out_ref[0] += contrib

        @pl.when(kk == ck - 1)
        def _():
            out_ref[0] = out_ref[0] * lax.rsqrt(deg_ref[:, 0:1])

    return pl.pallas_call(
        body,
        grid=grid,
        in_specs=[
            pl.BlockSpec((BM, 128), lambda m, n, k: (m, k)),
            pl.BlockSpec((128, 128), lambda m, n, k: (k, n)),
            pl.BlockSpec((BM, 16), lambda m, n, k: (m, 0)),
        ],
        out_specs=pl.BlockSpec((1, BM, 128), lambda m, n, k: (n, m, 0)),
        out_shape=jax.ShapeDtypeStruct((cn, NP, 128), jnp.float32),
    )(x, w, deg)


def _tc_matmul(agg, w, b_prev, deg):
    k_in, d_out = w.shape
    ck, cn = k_in // 128, d_out // 128
    grid = (N // BM, cn, ck)

    def body(agg_ref, w_ref, b_ref, deg_ref, out_ref):
        kk = pl.program_id(2)
        dinv = lax.rsqrt(deg_ref[:, 0:1])
        h = _leaky(dinv * agg_ref[0] + b_ref[0, 0])
        contrib = jnp.dot(h, w_ref[...], preferred_element_type=jnp.float32)

        @pl.when(kk == 0)
        def _():
            out_ref[0] = jnp.zeros_like(out_ref[0])

        out_ref[0] += contrib

        @pl.when(kk == ck - 1)
        def _():
            out_ref[0] = out_ref[0] * dinv

    return pl.pallas_call(
        body,
        grid=grid,
        in_specs=[
            pl.BlockSpec((1, BM, 128), lambda m, n, k: (k, m, 0)),
            pl.BlockSpec((128, 128), lambda m, n, k: (k, n)),
            pl.BlockSpec((1, 1, 128), lambda m, n, k: (k, 0, 0)),
            pl.BlockSpec((BM, 16), lambda m, n, k: (m, 0)),
        ],
        out_specs=pl.BlockSpec((1, BM, 128), lambda m, n, k: (n, m, 0)),
        out_shape=jax.ShapeDtypeStruct((cn, NP, 128), jnp.float32),
    )(agg, w, b_prev.reshape(ck, 1, 128), deg)


def _tc_epilogue(agg, b, deg, d_out):
    cn = d_out // 128
    grid = (N // BM, cn)

    def body(agg_ref, b_ref, deg_ref, out_ref):
        out_ref[...] = lax.rsqrt(deg_ref[:, 0:1]) * agg_ref[0] + b_ref[0, 0]

    return pl.pallas_call(
        body,
        grid=grid,
        in_specs=[
            pl.BlockSpec((1, BM, 128), lambda m, n: (n, m, 0)),
            pl.BlockSpec((1, 1, 128), lambda m, n: (n, 0, 0)),
            pl.BlockSpec((BM, 16), lambda m, n: (m, 0)),
        ],
        out_specs=pl.BlockSpec((BM, 128), lambda m, n: (m, n)),
        out_shape=jax.ShapeDtypeStruct((N, d_out), jnp.float32),
    )(agg, b.reshape(cn, 1, 128), deg)


_deg_kernel = _sc_deg()
_agg4 = _sc_agg(4)
_agg2 = _sc_agg(2)


def kernel(x, edge_index, W_enc, b_enc, W_h0, b_h0, W_h1, b_h1, W_h2, b_h2,
           W_dec, b_dec):
    src = jnp.pad(edge_index[0].reshape(NS, EPT),
                  ((0, 0), (0, EPAD))).reshape(NS, NBLK, KB)
    dst = jnp.pad(edge_index[1].reshape(NS, EPT), ((0, 0), (0, EPAD)),
                  constant_values=NP - 8).reshape(NS, NBLK, KB)

    deg = _deg_kernel(dst)

    y = _tc_matmul_first(x, W_enc, deg)
    agg = _agg4(y.reshape(4 * NP, 128), src, dst)
    y = _tc_matmul(agg.reshape(4, NP, 128), W_h0, b_enc, deg)
    agg = _agg4(y.reshape(4 * NP, 128), src, dst)
    y = _tc_matmul(agg.reshape(4, NP, 128), W_h1, b_h0, deg)
    agg = _agg4(y.reshape(4 * NP, 128), src, dst)
    y = _tc_matmul(agg.reshape(4, NP, 128), W_h2, b_h1, deg)
    agg = _agg4(y.reshape(4 * NP, 128), src, dst)
    y = _tc_matmul(agg.reshape(4, NP, 128), W_dec, b_h2, deg)
    agg = _agg2(y.reshape(2 * NP, 128), src, dst)
    return _tc_epilogue(agg.reshape(2, NP, 128), b_dec, deg, 256)
